# baseline (speedup 1.0000x reference)
import functools

import jax
import jax.numpy as jnp
from jax import lax
from jax.experimental import pallas as pl
from jax.experimental.pallas import tpu as pltpu
from jax.experimental.pallas import tpu_sc as plsc

N_F = 26
VOCAB_SZ = 100000
VPAD = 102400
D = 16
B = 16384

NC = 2
NS = 16
NW = NC * NS
LANES = 16

BPW = B // NW
VPB = BPW // LANES
GRP = BPW // LANES

HALVES = ((0, 7, 7), (14, 6, 6))


def _tc_reduce_body(t0_ref, t1_ref, s0_ref, s1_ref):
    s0_ref[...] = jnp.sum(t0_ref[0, :, :], axis=0)
    s1_ref[...] = jnp.sum(t1_ref[0, :, :], axis=0)


def _make_sc_body(fbase, n0, n1, with_partial):
    nf = n0 + n1

    def body(*refs):
        if with_partial:
            (s0_hbm, s1_hbm, xt_hbm, alpha_hbm, part_hbm, out_hbm,
             xbuf, idx0, idx1, g0, g1, alo, pbuf, outb, sem) = refs
        else:
            (s0_hbm, s1_hbm, xt_hbm, alpha_hbm, out_hbm,
             xbuf, idx0, idx1, g0, g1, alo, pbuf, outb, sem) = refs
        wid = lax.axis_index("s") * NC + lax.axis_index("c")

        for fp in range(nf):
            pltpu.sync_copy(xt_hbm.at[fbase + fp, pl.ds(wid * BPW, BPW)],
                            xbuf.at[pl.ds(fp * BPW, BPW)])
        if with_partial:
            pltpu.sync_copy(part_hbm.at[pl.ds(wid * BPW, BPW)], pbuf)

        pltpu.sync_copy(alpha_hbm, alo)
        for j in range(2):
            a = alo[pl.ds(j * LANES, LANES)]
            e = jnp.exp(a + a)
            alo[pl.ds(j * LANES, LANES)] = (e - 1.0) / (e + 1.0)
        ta = [plsc.load_gather(alo, [jnp.full((LANES,), fbase + f, jnp.int32)])
              for f in range(nf)]

        def make_idx(idx_ref, xoff, nfl):
            def idx_body(k, carry):
                s = k * LANES
                fof = (k // VPB) * VPAD
                idx_ref[pl.ds(s, LANES)] = (
                    xbuf[pl.ds(xoff + s, LANES)] + fof)
                return carry
            lax.fori_loop(0, nfl * VPB, idx_body, 0)
        make_idx(idx0, 0, n0)
        make_idx(idx1, n0 * BPW, n1)

        c0 = pltpu.async_copy(s0_hbm.at[idx0], g0, sem)
        c1 = pltpu.async_copy(s1_hbm.at[idx1], g1, sem)
        c0.wait()
        c1.wait()

        def group_body(g, carry):
            o = g * LANES
            acc = g0[pl.ds(o, LANES)] * ta[0]
            for f in range(1, n0):
                acc = acc + g0[pl.ds(f * BPW + o, LANES)] * ta[f]
            for f in range(n1):
                acc = acc + g1[pl.ds(f * BPW + o, LANES)] * ta[n0 + f]
            if with_partial:
                acc = acc + pbuf[pl.ds(o, LANES)]
            outb[pl.ds(o, LANES)] = acc
            return carry
        lax.fori_loop(0, GRP, group_body, 0)

        pltpu.sync_copy(outb, out_hbm.at[pl.ds(wid * BPW, BPW)])
    return body


@jax.jit
def kernel(X, tables, alpha):
    tt = jnp.transpose(tables, (0, 2, 1))
    xt = jnp.transpose(X, (1, 0))

    s_tabs = []
    for fbase, n0, n1 in HALVES:
        s0, s1 = pl.pallas_call(
            _tc_reduce_body,
            grid=(n0,),
            in_specs=[
                pl.BlockSpec((1, D, VPAD),
                             lambda f, fb=fbase: (fb + f, 0, 0)),
                pl.BlockSpec((1, D, VPAD),
                             lambda f, fb=fbase, m=n0: (fb + m + f, 0, 0)),
            ],
            out_specs=[
                pl.BlockSpec((VPAD,), lambda f: (f,)),
                pl.BlockSpec((VPAD,), lambda f: (f,)),
            ],
            out_shape=[
                jax.ShapeDtypeStruct((n0 * VPAD,), jnp.float32),
                jax.ShapeDtypeStruct((n1 * VPAD,), jnp.float32),
            ],
        )(tt, tt)
        s_tabs.append((s0, s1))

    alpha_pad = jnp.pad(alpha, (0, 2 * LANES - N_F))

    mesh = plsc.VectorSubcoreMesh(core_axis_name="c", subcore_axis_name="s")
    cp = pltpu.CompilerParams(
        needs_layout_passes=False, use_tc_tiling_on_sc=False)

    def scratch(n0, n1):
        return [
            pltpu.VMEM(((n0 + n1) * BPW,), jnp.int32),
            pltpu.VMEM((n0 * BPW,), jnp.int32),
            pltpu.VMEM((n1 * BPW,), jnp.int32),
            pltpu.VMEM((n0 * BPW,), jnp.float32),
            pltpu.VMEM((n1 * BPW,), jnp.float32),
            pltpu.VMEM((2 * LANES,), jnp.float32),
            pltpu.VMEM((BPW,), jnp.float32),
            pltpu.VMEM((BPW,), jnp.float32),
            pltpu.SemaphoreType.DMA,
        ]

    (fb0, a0, a1), (fb1, b0, b1) = HALVES
    part = pl.kernel(
        _make_sc_body(fb0, a0, a1, False),
        out_type=jax.ShapeDtypeStruct((B,), jnp.float32),
        mesh=mesh, compiler_params=cp, scratch_types=scratch(a0, a1),
    )(s_tabs[0][0], s_tabs[0][1], xt, alpha_pad)
    out = pl.kernel(
        _make_sc_body(fb1, b0, b1, True),
        out_type=jax.ShapeDtypeStruct((B,), jnp.float32),
        mesh=mesh, compiler_params=cp, scratch_types=scratch(b0, b1),
    )(s_tabs[1][0], s_tabs[1][1], xt, alpha_pad, part)
    return out[:, None]


# speedup vs baseline: 1.0953x; 1.0953x over previous
import functools

import jax
import jax.numpy as jnp
from jax import lax
from jax.experimental import pallas as pl
from jax.experimental.pallas import tpu as pltpu
from jax.experimental.pallas import tpu_sc as plsc

N_F = 26
VOCAB_SZ = 100000
VPAD = 102400
D = 16
B = 16384

NC = 2
NS = 16
NW = NC * NS
LANES = 16

BPW = B // NW
VPB = BPW // LANES
GRP = BPW // LANES

HALVES = ((0, 7, 7), (14, 6, 6))


def _tc_reduce_body(t0_ref, t1_ref, s0_ref, s1_ref):
    s0_ref[...] = jnp.sum(t0_ref[0, :, :], axis=0)
    s1_ref[...] = jnp.sum(t1_ref[0, :, :], axis=0)


def _prep_body(xt_hbm, i0_hbm, i1_hbm, i2_hbm, i3_hbm, xbuf, idxb, sem):
    wid = lax.axis_index("s") * NC + lax.axis_index("c")
    outs = [i0_hbm, i1_hbm, i2_hbm, i3_hbm]
    streams = []
    for (fb, n0, n1) in HALVES:
        streams += [(fb, n0), (fb + n0, n1)]
    for si, (fbase, nf) in enumerate(streams):
        for fp in range(nf):
            pltpu.sync_copy(xt_hbm.at[fbase + fp, pl.ds(wid * BPW, BPW)],
                            xbuf.at[pl.ds(fp * BPW, BPW)])

        def idx_body(k, carry, nf=nf):
            s = k * LANES
            fof = (k // VPB) * VPAD
            idxb[pl.ds(s, LANES)] = xbuf[pl.ds(s, LANES)] + fof
            return carry
        lax.fori_loop(0, nf * VPB, idx_body, 0)
        pltpu.sync_copy(idxb.at[pl.ds(0, nf * BPW)],
                        outs[si].at[pl.ds(wid * nf * BPW, nf * BPW)])


def _make_gather_body(fbase, n0, n1, with_partial):
    nf = n0 + n1

    def body(*refs):
        if with_partial:
            (s0_hbm, s1_hbm, i0_hbm, i1_hbm, alpha_hbm, part_hbm, out_hbm,
             idx0, idx1, g0, g1, alo, pbuf, outb, sem) = refs
        else:
            (s0_hbm, s1_hbm, i0_hbm, i1_hbm, alpha_hbm, out_hbm,
             idx0, idx1, g0, g1, alo, pbuf, outb, sem) = refs
        wid = lax.axis_index("s") * NC + lax.axis_index("c")

        pltpu.sync_copy(i0_hbm.at[pl.ds(wid * n0 * BPW, n0 * BPW)], idx0)
        pltpu.sync_copy(i1_hbm.at[pl.ds(wid * n1 * BPW, n1 * BPW)], idx1)
        c0 = pltpu.async_copy(s0_hbm.at[idx0], g0, sem)
        c1 = pltpu.async_copy(s1_hbm.at[idx1], g1, sem)

        if with_partial:
            pltpu.sync_copy(part_hbm.at[pl.ds(wid * BPW, BPW)], pbuf)
        pltpu.sync_copy(alpha_hbm, alo)
        for j in range(2):
            a = alo[pl.ds(j * LANES, LANES)]
            e = jnp.exp(a + a)
            alo[pl.ds(j * LANES, LANES)] = (e - 1.0) / (e + 1.0)
        ta = [plsc.load_gather(alo, [jnp.full((LANES,), fbase + f, jnp.int32)])
              for f in range(nf)]

        c0.wait()
        c1.wait()

        def group_body(g, carry):
            o = g * LANES
            acc = g0[pl.ds(o, LANES)] * ta[0]
            for f in range(1, n0):
                acc = acc + g0[pl.ds(f * BPW + o, LANES)] * ta[f]
            for f in range(n1):
                acc = acc + g1[pl.ds(f * BPW + o, LANES)] * ta[n0 + f]
            if with_partial:
                acc = acc + pbuf[pl.ds(o, LANES)]
            outb[pl.ds(o, LANES)] = acc
            return carry
        lax.fori_loop(0, GRP, group_body, 0)

        pltpu.sync_copy(outb, out_hbm.at[pl.ds(wid * BPW, BPW)])
    return body


@jax.jit
def kernel(X, tables, alpha):
    tt = jnp.transpose(tables, (0, 2, 1))
    xt = jnp.transpose(X, (1, 0))

    mesh = plsc.VectorSubcoreMesh(core_axis_name="c", subcore_axis_name="s")
    cp = pltpu.CompilerParams(
        needs_layout_passes=False, use_tc_tiling_on_sc=False)

    streams = []
    for (fb, n0, n1) in HALVES:
        streams += [(fb, n0), (fb + n0, n1)]

    idx_lists = pl.kernel(
        _prep_body,
        out_type=[jax.ShapeDtypeStruct((NW * nf * BPW,), jnp.int32)
                  for (_, nf) in streams],
        mesh=mesh, compiler_params=cp,
        scratch_types=[
            pltpu.VMEM((7 * BPW,), jnp.int32),
            pltpu.VMEM((7 * BPW,), jnp.int32),
            pltpu.SemaphoreType.DMA,
        ],
    )(xt)

    s_tabs = []
    for fbase, n0, n1 in HALVES:
        s0, s1 = pl.pallas_call(
            _tc_reduce_body,
            grid=(n0,),
            in_specs=[
                pl.BlockSpec((1, D, VPAD),
                             lambda f, fb=fbase: (fb + f, 0, 0)),
                pl.BlockSpec((1, D, VPAD),
                             lambda f, fb=fbase, m=n0: (fb + m + f, 0, 0)),
            ],
            out_specs=[
                pl.BlockSpec((VPAD,), lambda f: (f,)),
                pl.BlockSpec((VPAD,), lambda f: (f,)),
            ],
            out_shape=[
                jax.ShapeDtypeStruct((n0 * VPAD,), jnp.float32),
                jax.ShapeDtypeStruct((n1 * VPAD,), jnp.float32),
            ],
        )(tt, tt)
        s_tabs.append((s0, s1))

    alpha_pad = jnp.pad(alpha, (0, 2 * LANES - N_F))

    def scratch(n0, n1):
        return [
            pltpu.VMEM((n0 * BPW,), jnp.int32),
            pltpu.VMEM((n1 * BPW,), jnp.int32),
            pltpu.VMEM((n0 * BPW,), jnp.float32),
            pltpu.VMEM((n1 * BPW,), jnp.float32),
            pltpu.VMEM((2 * LANES,), jnp.float32),
            pltpu.VMEM((BPW,), jnp.float32),
            pltpu.VMEM((BPW,), jnp.float32),
            pltpu.SemaphoreType.DMA,
        ]

    (fb0, a0, a1), (fb1, b0, b1) = HALVES
    part = pl.kernel(
        _make_gather_body(fb0, a0, a1, False),
        out_type=jax.ShapeDtypeStruct((B,), jnp.float32),
        mesh=mesh, compiler_params=cp, scratch_types=scratch(a0, a1),
    )(s_tabs[0][0], s_tabs[0][1], idx_lists[0], idx_lists[1], alpha_pad)
    out = pl.kernel(
        _make_gather_body(fb1, b0, b1, True),
        out_type=jax.ShapeDtypeStruct((B,), jnp.float32),
        mesh=mesh, compiler_params=cp, scratch_types=scratch(b0, b1),
    )(s_tabs[1][0], s_tabs[1][1], idx_lists[2], idx_lists[3], alpha_pad, part)
    return out[:, None]


# speedup vs baseline: 1.1031x; 1.0071x over previous
import functools

import jax
import jax.numpy as jnp
from jax import lax
from jax.experimental import pallas as pl
from jax.experimental.pallas import tpu as pltpu
from jax.experimental.pallas import tpu_sc as plsc

N_F = 26
VOCAB_SZ = 100000
VPAD = 102400
D = 16
B = 16384

NC = 2
NS = 16
NW = NC * NS
LANES = 16

BPW = B // NW
VPB = BPW // LANES
GRP = BPW // LANES

HALVES = ((0, 8, 8), (16, 5, 5))


def _tc_reduce_body(t0_ref, t1_ref, s0_ref, s1_ref):
    s0_ref[...] = jnp.sum(t0_ref[0, :, :], axis=0)
    s1_ref[...] = jnp.sum(t1_ref[0, :, :], axis=0)


def _prep_body(xt_hbm, i0_hbm, i1_hbm, i2_hbm, i3_hbm, xbuf, idxb, sem):
    wid = lax.axis_index("s") * NC + lax.axis_index("c")
    outs = [i0_hbm, i1_hbm, i2_hbm, i3_hbm]
    pltpu.sync_copy(xt_hbm.at[pl.ds(0, N_F), pl.ds(wid * BPW, BPW)], xbuf)
    streams = []
    for (fb, n0, n1) in HALVES:
        streams += [(fb, n0), (fb + n0, n1)]
    for si, (fbase, nf) in enumerate(streams):
        def idx_body(k, carry, fbase=fbase):
            fl = k // VPB
            s = (k - fl * VPB) * LANES
            idxb[pl.ds(k * LANES, LANES)] = (
                xbuf[fbase + fl, pl.ds(s, LANES)] + fl * VPAD)
            return carry
        lax.fori_loop(0, nf * VPB, idx_body, 0)
        pltpu.sync_copy(idxb.at[pl.ds(0, nf * BPW)],
                        outs[si].at[pl.ds(wid * nf * BPW, nf * BPW)])


def _make_gather_body(fbase, n0, n1, with_partial):
    nf = n0 + n1

    def body(*refs):
        if with_partial:
            (s0_hbm, s1_hbm, i0_hbm, i1_hbm, alpha_hbm, part_hbm, out_hbm,
             idx0, idx1, g0, g1, alo, pbuf, outb, sem) = refs
        else:
            (s0_hbm, s1_hbm, i0_hbm, i1_hbm, alpha_hbm, out_hbm,
             idx0, idx1, g0, g1, alo, pbuf, outb, sem) = refs
        wid = lax.axis_index("s") * NC + lax.axis_index("c")

        pltpu.sync_copy(i0_hbm.at[pl.ds(wid * n0 * BPW, n0 * BPW)], idx0)
        pltpu.sync_copy(i1_hbm.at[pl.ds(wid * n1 * BPW, n1 * BPW)], idx1)
        c0 = pltpu.async_copy(s0_hbm.at[idx0], g0, sem)
        c1 = pltpu.async_copy(s1_hbm.at[idx1], g1, sem)

        if with_partial:
            pltpu.sync_copy(part_hbm.at[pl.ds(wid * BPW, BPW)], pbuf)
        pltpu.sync_copy(alpha_hbm, alo)
        for j in range(2):
            a = alo[pl.ds(j * LANES, LANES)]
            e = jnp.exp(a + a)
            alo[pl.ds(j * LANES, LANES)] = (e - 1.0) / (e + 1.0)
        ta = [plsc.load_gather(alo, [jnp.full((LANES,), fbase + f, jnp.int32)])
              for f in range(nf)]

        c0.wait()
        c1.wait()

        def group_body(g, carry):
            o = g * LANES
            acc = g0[pl.ds(o, LANES)] * ta[0]
            for f in range(1, n0):
                acc = acc + g0[pl.ds(f * BPW + o, LANES)] * ta[f]
            for f in range(n1):
                acc = acc + g1[pl.ds(f * BPW + o, LANES)] * ta[n0 + f]
            if with_partial:
                acc = acc + pbuf[pl.ds(o, LANES)]
            outb[pl.ds(o, LANES)] = acc
            return carry
        lax.fori_loop(0, GRP, group_body, 0)

        pltpu.sync_copy(outb, out_hbm.at[pl.ds(wid * BPW, BPW)])
    return body


@jax.jit
def kernel(X, tables, alpha):
    tt = jnp.transpose(tables, (0, 2, 1))
    xt = jnp.transpose(X, (1, 0))

    mesh = plsc.VectorSubcoreMesh(core_axis_name="c", subcore_axis_name="s")
    cp = pltpu.CompilerParams(
        needs_layout_passes=False, use_tc_tiling_on_sc=False)

    streams = []
    for (fb, n0, n1) in HALVES:
        streams += [(fb, n0), (fb + n0, n1)]

    idx_lists = pl.kernel(
        _prep_body,
        out_type=[jax.ShapeDtypeStruct((NW * nf * BPW,), jnp.int32)
                  for (_, nf) in streams],
        mesh=mesh, compiler_params=cp,
        scratch_types=[
            pltpu.VMEM((N_F, BPW), jnp.int32),
            pltpu.VMEM((8 * BPW,), jnp.int32),
            pltpu.SemaphoreType.DMA,
        ],
    )(xt)

    s_tabs = []
    for fbase, n0, n1 in HALVES:
        s0, s1 = pl.pallas_call(
            _tc_reduce_body,
            grid=(n0,),
            in_specs=[
                pl.BlockSpec((1, D, VPAD),
                             lambda f, fb=fbase: (fb + f, 0, 0)),
                pl.BlockSpec((1, D, VPAD),
                             lambda f, fb=fbase, m=n0: (fb + m + f, 0, 0)),
            ],
            out_specs=[
                pl.BlockSpec((VPAD,), lambda f: (f,)),
                pl.BlockSpec((VPAD,), lambda f: (f,)),
            ],
            out_shape=[
                jax.ShapeDtypeStruct((n0 * VPAD,), jnp.float32),
                jax.ShapeDtypeStruct((n1 * VPAD,), jnp.float32),
            ],
        )(tt, tt)
        s_tabs.append((s0, s1))

    alpha_pad = jnp.pad(alpha, (0, 2 * LANES - N_F))

    def scratch(n0, n1):
        return [
            pltpu.VMEM((n0 * BPW,), jnp.int32),
            pltpu.VMEM((n1 * BPW,), jnp.int32),
            pltpu.VMEM((n0 * BPW,), jnp.float32),
            pltpu.VMEM((n1 * BPW,), jnp.float32),
            pltpu.VMEM((2 * LANES,), jnp.float32),
            pltpu.VMEM((BPW,), jnp.float32),
            pltpu.VMEM((BPW,), jnp.float32),
            pltpu.SemaphoreType.DMA,
        ]

    (fb0, a0, a1), (fb1, b0, b1) = HALVES
    part = pl.kernel(
        _make_gather_body(fb0, a0, a1, False),
        out_type=jax.ShapeDtypeStruct((B,), jnp.float32),
        mesh=mesh, compiler_params=cp, scratch_types=scratch(a0, a1),
    )(s_tabs[0][0], s_tabs[0][1], idx_lists[0], idx_lists[1], alpha_pad)
    out = pl.kernel(
        _make_gather_body(fb1, b0, b1, True),
        out_type=jax.ShapeDtypeStruct((B,), jnp.float32),
        mesh=mesh, compiler_params=cp, scratch_types=scratch(b0, b1),
    )(s_tabs[1][0], s_tabs[1][1], idx_lists[2], idx_lists[3], alpha_pad, part)
    return out[:, None]


# speedup vs baseline: 1.1046x; 1.0014x over previous
import jax
import jax.numpy as jnp
from jax import lax
from jax.experimental import pallas as pl
from jax.experimental.pallas import tpu as pltpu
from jax.experimental.pallas import tpu_sc as plsc

N_F = 26
VOCAB_SZ = 100000
VPAD = 102400
D = 16
B = 16384

NC = 2
NS = 16
NW = NC * NS
LANES = 16

BPW = B // NW
VPB = BPW // LANES
GRP = BPW // LANES

HALVES = ((0, 8, 8), (16, 5, 5))


def _tc_reduce_body(t0_ref, t1_ref, s0_ref, s1_ref):
    s0_ref[...] = jnp.sum(t0_ref[0, :, :], axis=0)
    s1_ref[...] = jnp.sum(t1_ref[0, :, :], axis=0)


def _prep_body(xt_hbm, i0_hbm, i1_hbm, i2_hbm, i3_hbm, xbuf, idxb, sem):
    wid = lax.axis_index("s") * NC + lax.axis_index("c")
    outs = [i0_hbm, i1_hbm, i2_hbm, i3_hbm]
    pltpu.sync_copy(xt_hbm.at[pl.ds(0, N_F), pl.ds(wid * BPW, BPW)], xbuf)
    streams = []
    for (fb, n0, n1) in HALVES:
        streams += [(fb, n0), (fb + n0, n1)]
    for si, (fbase, nf) in enumerate(streams):
        def idx_body(k, carry, fbase=fbase):
            fl = k // VPB
            s = (k - fl * VPB) * LANES
            idxb[pl.ds(k * LANES, LANES)] = (
                xbuf[fbase + fl, pl.ds(s, LANES)] + fl * VPAD)
            return carry
        lax.fori_loop(0, nf * VPB, idx_body, 0)
        pltpu.sync_copy(idxb.at[pl.ds(0, nf * BPW)],
                        outs[si].at[pl.ds(wid * nf * BPW, nf * BPW)])


def _make_gather_body(fbase, n0, n1, with_partial):
    nf = n0 + n1

    def body(*refs):
        if with_partial:
            (s0_hbm, s1_hbm, i0_hbm, i1_hbm, alpha_hbm, part_hbm, out_hbm,
             idx0, idx1, g0, g1, alo, pbuf, outb, sem) = refs
        else:
            (s0_hbm, s1_hbm, i0_hbm, i1_hbm, alpha_hbm, out_hbm,
             idx0, idx1, g0, g1, alo, pbuf, outb, sem) = refs
        wid = lax.axis_index("s") * NC + lax.axis_index("c")

        pltpu.sync_copy(i0_hbm.at[pl.ds(wid * n0 * BPW, n0 * BPW)], idx0)
        pltpu.sync_copy(i1_hbm.at[pl.ds(wid * n1 * BPW, n1 * BPW)], idx1)
        c0 = pltpu.async_copy(s0_hbm.at[idx0], g0, sem)
        c1 = pltpu.async_copy(s1_hbm.at[idx1], g1, sem)

        if with_partial:
            pltpu.sync_copy(part_hbm.at[pl.ds(wid * BPW, BPW)], pbuf)
        pltpu.sync_copy(alpha_hbm, alo)
        for j in range(2):
            a = alo[pl.ds(j * LANES, LANES)]
            e = jnp.exp(a + a)
            alo[pl.ds(j * LANES, LANES)] = (e - 1.0) / (e + 1.0)
        ta = [plsc.load_gather(alo, [jnp.full((LANES,), fbase + f, jnp.int32)])
              for f in range(nf)]

        c0.wait()
        c1.wait()

        def group_body(g, carry):
            o = g * LANES
            acc = g0[pl.ds(o, LANES)] * ta[0]
            for f in range(1, n0):
                acc = acc + g0[pl.ds(f * BPW + o, LANES)] * ta[f]
            for f in range(n1):
                acc = acc + g1[pl.ds(f * BPW + o, LANES)] * ta[n0 + f]
            if with_partial:
                acc = acc + pbuf[pl.ds(o, LANES)]
            outb[pl.ds(o, LANES)] = acc
            return carry
        lax.fori_loop(0, GRP, group_body, 0)

        pltpu.sync_copy(outb, out_hbm.at[pl.ds(wid * BPW, BPW)])
    return body


@jax.jit
def kernel(X, tables, alpha):
    tt = jnp.transpose(tables, (0, 2, 1))
    xt = jnp.transpose(X, (1, 0))

    mesh = plsc.VectorSubcoreMesh(core_axis_name="c", subcore_axis_name="s")
    cp = pltpu.CompilerParams(
        needs_layout_passes=False, use_tc_tiling_on_sc=False)

    streams = []
    for (fb, n0, n1) in HALVES:
        streams += [(fb, n0), (fb + n0, n1)]

    idx_lists = pl.kernel(
        _prep_body,
        out_type=[jax.ShapeDtypeStruct((NW * nf * BPW,), jnp.int32)
                  for (_, nf) in streams],
        mesh=mesh, compiler_params=cp,
        scratch_types=[
            pltpu.VMEM((N_F, BPW), jnp.int32),
            pltpu.VMEM((8 * BPW,), jnp.int32),
            pltpu.SemaphoreType.DMA,
        ],
    )(xt)

    s_tabs = []
    for fbase, n0, n1 in HALVES:
        s0, s1 = pl.pallas_call(
            _tc_reduce_body,
            grid=(n0,),
            in_specs=[
                pl.BlockSpec((1, D, VPAD),
                             lambda f, fb=fbase: (fb + f, 0, 0)),
                pl.BlockSpec((1, D, VPAD),
                             lambda f, fb=fbase, m=n0: (fb + m + f, 0, 0)),
            ],
            out_specs=[
                pl.BlockSpec((VPAD,), lambda f: (f,)),
                pl.BlockSpec((VPAD,), lambda f: (f,)),
            ],
            out_shape=[
                jax.ShapeDtypeStruct((n0 * VPAD,), jnp.float32),
                jax.ShapeDtypeStruct((n1 * VPAD,), jnp.float32),
            ],
        )(tt, tt)
        s_tabs.append((s0, s1))

    alpha_pad = jnp.pad(alpha, (0, 2 * LANES - N_F))

    def scratch(n0, n1):
        return [
            pltpu.VMEM((n0 * BPW,), jnp.int32),
            pltpu.VMEM((n1 * BPW,), jnp.int32),
            pltpu.VMEM((n0 * BPW,), jnp.float32),
            pltpu.VMEM((n1 * BPW,), jnp.float32),
            pltpu.VMEM((2 * LANES,), jnp.float32),
            pltpu.VMEM((BPW,), jnp.float32),
            pltpu.VMEM((BPW,), jnp.float32),
            pltpu.SemaphoreType.DMA,
        ]

    (fb0, a0, a1), (fb1, b0, b1) = HALVES
    part = pl.kernel(
        _make_gather_body(fb0, a0, a1, False),
        out_type=jax.ShapeDtypeStruct((B,), jnp.float32),
        mesh=mesh, compiler_params=cp, scratch_types=scratch(a0, a1),
    )(s_tabs[0][0], s_tabs[0][1], idx_lists[0], idx_lists[1], alpha_pad)
    out = pl.kernel(
        _make_gather_body(fb1, b0, b1, True),
        out_type=jax.ShapeDtypeStruct((B,), jnp.float32),
        mesh=mesh, compiler_params=cp, scratch_types=scratch(b0, b1),
    )(s_tabs[1][0], s_tabs[1][1], idx_lists[2], idx_lists[3], alpha_pad, part)
    return out[:, None]
